# Initial kernel scaffold; baseline (speedup 1.0000x reference)
import jax
import jax.numpy as jnp
from jax.experimental import pallas as pl


def kernel(words, pos_tags, word_table, pos_table):
    raise NotImplementedError("write your pallas kernel here")


# speedup vs baseline: 1.4699x; 1.4699x over previous
import functools

import jax
import jax.numpy as jnp
from jax import lax
from jax.experimental import pallas as pl
from jax.experimental.pallas import tpu as pltpu
from jax.experimental.pallas import tpu_sc as plsc

_VOCAB = 100000
_D = 50
_DP = 56
_B = 4096
_L = 200

_info = plsc.get_sparse_core_info()
_NC, _NS = _info.num_cores, _info.num_subcores
_NW = _NC * _NS
_ROWS = 2 * _B * _L
_PER_W = _ROWS // _NW
_CH = 256
_NCHUNK = _PER_W // _CH
_NBUF = 4
_PRE = 2


@functools.partial(
    pl.kernel,
    mesh=plsc.VectorSubcoreMesh(core_axis_name="c", subcore_axis_name="s"),
    out_type=jax.ShapeDtypeStruct((_ROWS, _DP), jnp.float32),
    scratch_types=(
        [pltpu.VMEM((_NCHUNK, _CH), jnp.int32)]
        + [pltpu.VMEM((_CH, _DP), jnp.float32) for _ in range(_NBUF)]
        + [pltpu.SemaphoreType.DMA for _ in range(_NBUF)]
    ),
    compiler_params=pltpu.CompilerParams(use_tc_tiling_on_sc=False),
)
def _gather_rows(table_hbm, idx_hbm, out_hbm, idx_v, *bufs_and_sems):
    bufs = bufs_and_sems[:_NBUF]
    sems = bufs_and_sems[_NBUF:]
    wid = lax.axis_index("s") * _NC + lax.axis_index("c")
    base = wid * _PER_W
    pltpu.sync_copy(idx_hbm.at[pl.ds(wid * _NCHUNK, _NCHUNK)], idx_v)

    def start_gather(g, b):
        pltpu.async_copy(table_hbm.at[idx_v.at[g]], bufs[b], sems[b])

    def wait_gather(b):
        pltpu.make_async_copy(
            table_hbm.at[pl.ds(0, _CH)], bufs[b], sems[b]
        ).wait()

    for d in range(_PRE):
        start_gather(d, d)

    def group(t, carry):
        for b in range(_NBUF):
            g = t * _NBUF + b
            h = g + _PRE

            @pl.when(h < _NCHUNK)
            def _():
                start_gather(h, (b + _PRE) % _NBUF)

            wait_gather(b)
            pltpu.sync_copy(bufs[b], out_hbm.at[pl.ds(base + g * _CH, _CH)])
        return carry

    lax.fori_loop(0, _NCHUNK // _NBUF, group, 0)


def kernel(words, pos_tags, word_table, pos_table):
    table = jnp.concatenate([word_table, pos_table], axis=0)
    table = jnp.pad(table, ((0, 0), (0, _DP - _D)))
    idx = jnp.stack(
        [words.reshape(-1).astype(jnp.int32),
         pos_tags.reshape(-1).astype(jnp.int32) + _VOCAB],
        axis=-1,
    ).reshape(_ROWS // _CH, _CH)
    out = _gather_rows(table, idx)
    return out[:, :_D].reshape(_B, _L, 2 * _D)


# speedup vs baseline: 1.9440x; 1.3225x over previous
import functools

import jax
import jax.numpy as jnp
from jax import lax
from jax.experimental import pallas as pl
from jax.experimental.pallas import tpu as pltpu
from jax.experimental.pallas import tpu_sc as plsc

_VOCAB = 100000
_D = 50
_B = 4096
_L = 200

_info = plsc.get_sparse_core_info()
_NC, _NS = _info.num_cores, _info.num_subcores
_NW = _NC * _NS
_ROWS = 2 * _B * _L
_BPW = _B // _NW
_CH = 2 * _L
_NCHUNK = _BPW
_NBUF = 4
_PRE = 2
_NV = 13


@functools.partial(
    pl.kernel,
    mesh=plsc.VectorSubcoreMesh(core_axis_name="c", subcore_axis_name="s"),
    out_type=jax.ShapeDtypeStruct((_ROWS, _D), jnp.float32),
    scratch_types=(
        [pltpu.VMEM((_D, _D), jnp.float32)]
        + [pltpu.VMEM((_L,), jnp.int32) for _ in range(_NBUF)]
        + [pltpu.VMEM((_L,), jnp.int32) for _ in range(_NBUF)]
        + [pltpu.VMEM((_CH,), jnp.int32) for _ in range(_NBUF)]
        + [pltpu.VMEM((_L,), jnp.int32) for _ in range(_NBUF)]
        + [pltpu.VMEM((_CH, 48), jnp.float32) for _ in range(_NBUF)]
        + [pltpu.VMEM((_L, 8), jnp.float32) for _ in range(_NBUF)]
        + [pltpu.VMEM((_CH, _D), jnp.float32)]
        + [pltpu.SemaphoreType.DMA for _ in range(4 * _NBUF)]
    ),
    compiler_params=pltpu.CompilerParams(
        use_tc_tiling_on_sc=False, needs_layout_passes=False),
)
def _emb_kernel(t48_hbm, wf8_hbm, words_hbm, pos_hbm, pos_tab_hbm, out_hbm,
                postab, *scr):
    rawW = scr[0:_NBUF]
    rawP = scr[_NBUF:2 * _NBUF]
    idxb = scr[2 * _NBUF:3 * _NBUF]
    twb = scr[3 * _NBUF:4 * _NBUF]
    b48 = scr[4 * _NBUF:5 * _NBUF]
    b8w = scr[5 * _NBUF:6 * _NBUF]
    b50 = scr[6 * _NBUF]
    sems = scr[6 * _NBUF + 1:]
    isemW = sems[0:_NBUF]
    isemP = sems[_NBUF:2 * _NBUF]
    sem48 = sems[2 * _NBUF:3 * _NBUF]
    semw = sems[3 * _NBUF:]

    wid = lax.axis_index("s") * _NC + lax.axis_index("c")
    row0 = wid * _BPW

    pltpu.sync_copy(pos_tab_hbm, postab)

    iota = lax.iota(jnp.int32, 16)
    c48 = iota * 0 + 48
    c49 = iota * 0 + 49

    def fetch_raw(g, b):
        pltpu.async_copy(words_hbm.at[row0 + g], rawW[b], isemW[b])
        pltpu.async_copy(pos_hbm.at[row0 + g], rawP[b], isemP[b])

    def wait_raw(b):
        pltpu.make_async_copy(words_hbm.at[0], rawW[b], isemW[b]).wait()
        pltpu.make_async_copy(pos_hbm.at[0], rawP[b], isemP[b]).wait()

    def build_idx(b):
        for v in range(_NV):
            t = iota + v * 16
            m = t < _L
            w = plsc.load_gather(rawW[b], [t], mask=m)
            p = plsc.load_gather(rawP[b], [t], mask=m)
            plsc.store_scatter(idxb[b], [2 * t], w, mask=m)
            plsc.store_scatter(idxb[b], [2 * t + 1], p + _VOCAB, mask=m)
            dw = lax.shift_right_logical(w * 50 + 48, iota * 0 + 3)
            plsc.store_scatter(twb[b], [t], dw, mask=m)

    def start_gathers(b):
        pltpu.async_copy(t48_hbm.at[idxb[b]], b48[b], sem48[b])
        pltpu.async_copy(wf8_hbm.at[twb[b]], b8w[b], semw[b])

    def wait_gathers(b):
        pltpu.make_async_copy(t48_hbm.at[pl.ds(0, _CH)], b48[b],
                              sem48[b]).wait()
        pltpu.make_async_copy(wf8_hbm.at[pl.ds(0, _L)], b8w[b],
                              semw[b]).wait()

    def tail_fix(b):
        for v in range(_NV):
            t = iota + v * 16
            m = t < _L
            w = plsc.load_gather(rawW[b], [t], mask=m)
            o = lax.bitwise_and(w * 2, iota * 0 + 7)
            x0 = plsc.load_gather(b8w[b], [t, o], mask=m)
            x1 = plsc.load_gather(b8w[b], [t, o + 1], mask=m)
            plsc.store_scatter(b50, [2 * t, c48], x0, mask=m)
            plsc.store_scatter(b50, [2 * t, c49], x1, mask=m)
            p = plsc.load_gather(rawP[b], [t], mask=m)
            u0 = plsc.load_gather(postab, [p, c48], mask=m)
            u1 = plsc.load_gather(postab, [p, c49], mask=m)
            plsc.store_scatter(b50, [2 * t + 1, c48], u0, mask=m)
            plsc.store_scatter(b50, [2 * t + 1, c49], u1, mask=m)

    for d in range(_NBUF):
        fetch_raw(d, d)
    for d in range(_PRE):
        wait_raw(d)
        build_idx(d)
        start_gathers(d)

    def group(tt, carry):
        for b in range(_NBUF):
            g = tt * _NBUF + b
            h = g + _PRE
            bh = (b + _PRE) % _NBUF

            @pl.when(h < _NCHUNK)
            def _():
                wait_raw(bh)
                build_idx(bh)
                start_gathers(bh)

            wait_gathers(b)

            def rowcp(r, carry):
                for k in range(3):
                    b50[r, pl.ds(16 * k, 16)] = b48[b][r, pl.ds(16 * k, 16)]
                return carry

            lax.fori_loop(0, _CH, rowcp, 0)
            tail_fix(b)
            pltpu.sync_copy(b50,
                            out_hbm.at[pl.ds((row0 + g) * _CH, _CH)])

            @pl.when(g + _NBUF < _NCHUNK)
            def _():
                fetch_raw(g + _NBUF, b)
        return carry

    lax.fori_loop(0, _NCHUNK // _NBUF, group, 0)


def kernel(words, pos_tags, word_table, pos_table):
    t48 = jnp.concatenate([word_table[:, :48], pos_table[:, :48]], axis=0)
    wf8 = word_table.reshape(_VOCAB * _D // 8, 8)
    out = _emb_kernel(t48, wf8, words.astype(jnp.int32),
                      pos_tags.astype(jnp.int32), pos_table)
    return out.reshape(_B, _L, 2 * _D)


# speedup vs baseline: 2.7748x; 1.4274x over previous
import functools

import jax
import jax.numpy as jnp
from jax import lax
from jax.experimental import pallas as pl
from jax.experimental.pallas import tpu as pltpu
from jax.experimental.pallas import tpu_sc as plsc

_VOCAB = 100000
_D = 50
_B = 4096
_L = 200

_info = plsc.get_sparse_core_info()
_NC, _NS = _info.num_cores, _info.num_subcores
_NW = _NC * _NS
_ROWS = 2 * _B * _L
_BPW = _B // _NW
_CH = 2 * _L
_NCHUNK = _BPW
_NBUF = 4
_PRE = 2
_NV = 13


@functools.partial(
    pl.kernel,
    mesh=plsc.VectorSubcoreMesh(core_axis_name="c", subcore_axis_name="s"),
    out_type=jax.ShapeDtypeStruct((_B, _L, 2 * _D), jnp.float32),
    scratch_types=(
        [pltpu.VMEM((_D, _D), jnp.float32)]
        + [pltpu.VMEM((_L,), jnp.int32) for _ in range(_NBUF)]
        + [pltpu.VMEM((_L,), jnp.int32) for _ in range(_NBUF)]
        + [pltpu.VMEM((_CH,), jnp.int32) for _ in range(_NBUF)]
        + [pltpu.VMEM((_L,), jnp.int32) for _ in range(_NBUF)]
        + [pltpu.VMEM((_CH, 48), jnp.float32) for _ in range(_NBUF)]
        + [pltpu.VMEM((_L, 8), jnp.float32) for _ in range(_NBUF)]
        + [pltpu.VMEM((_L, 2 * _D), jnp.float32)]
        + [pltpu.SemaphoreType.DMA for _ in range(4 * _NBUF)]
    ),
    compiler_params=pltpu.CompilerParams(
        use_tc_tiling_on_sc=False, needs_layout_passes=False),
)
def _emb_kernel(t48_hbm, wf8_hbm, words_hbm, pos_hbm, pos_tab_hbm, out_hbm,
                postab, *scr):
    rawW = scr[0:_NBUF]
    rawP = scr[_NBUF:2 * _NBUF]
    idxb = scr[2 * _NBUF:3 * _NBUF]
    twb = scr[3 * _NBUF:4 * _NBUF]
    b48 = scr[4 * _NBUF:5 * _NBUF]
    b8w = scr[5 * _NBUF:6 * _NBUF]
    b50 = scr[6 * _NBUF]
    sems = scr[6 * _NBUF + 1:]
    isemW = sems[0:_NBUF]
    isemP = sems[_NBUF:2 * _NBUF]
    sem48 = sems[2 * _NBUF:3 * _NBUF]
    semw = sems[3 * _NBUF:]

    wid = lax.axis_index("s") * _NC + lax.axis_index("c")
    row0 = wid * _BPW

    pltpu.sync_copy(pos_tab_hbm, postab)

    iota = lax.iota(jnp.int32, 16)
    c48 = iota * 0 + 48
    c49 = iota * 0 + 49
    c98 = iota * 0 + 98
    c99 = iota * 0 + 99

    def fetch_raw(g, b):
        pltpu.async_copy(words_hbm.at[row0 + g], rawW[b], isemW[b])
        pltpu.async_copy(pos_hbm.at[row0 + g], rawP[b], isemP[b])

    def wait_raw(b):
        pltpu.make_async_copy(words_hbm.at[0], rawW[b], isemW[b]).wait()
        pltpu.make_async_copy(pos_hbm.at[0], rawP[b], isemP[b]).wait()

    def build_idx(b):
        for v in range(_NV):
            t = iota + v * 16
            m = t < _L
            w = plsc.load_gather(rawW[b], [t], mask=m)
            p = plsc.load_gather(rawP[b], [t], mask=m)
            plsc.store_scatter(idxb[b], [2 * t], w, mask=m)
            plsc.store_scatter(idxb[b], [2 * t + 1], p + _VOCAB, mask=m)
            dw = lax.shift_right_logical(w * 50 + 48, iota * 0 + 3)
            plsc.store_scatter(twb[b], [t], dw, mask=m)

    def start_gathers(b):
        pltpu.async_copy(t48_hbm.at[idxb[b]], b48[b], sem48[b])
        pltpu.async_copy(wf8_hbm.at[twb[b]], b8w[b], semw[b])

    def wait_gathers(b):
        pltpu.make_async_copy(t48_hbm.at[pl.ds(0, _CH)], b48[b],
                              sem48[b]).wait()
        pltpu.make_async_copy(wf8_hbm.at[pl.ds(0, _L)], b8w[b],
                              semw[b]).wait()

    def tail_fix(b):
        for v in range(_NV):
            t = iota + v * 16
            m = t < _L
            w = plsc.load_gather(rawW[b], [t], mask=m)
            o = lax.bitwise_and(w * 2, iota * 0 + 7)
            x0 = plsc.load_gather(b8w[b], [t, o], mask=m)
            x1 = plsc.load_gather(b8w[b], [t, o + 1], mask=m)
            plsc.store_scatter(b50, [t, c48], x0, mask=m)
            plsc.store_scatter(b50, [t, c49], x1, mask=m)
            p = plsc.load_gather(rawP[b], [t], mask=m)
            u0 = plsc.load_gather(postab, [p, c48], mask=m)
            u1 = plsc.load_gather(postab, [p, c49], mask=m)
            plsc.store_scatter(b50, [t, c98], u0, mask=m)
            plsc.store_scatter(b50, [t, c99], u1, mask=m)

    for d in range(_NBUF):
        fetch_raw(d, d)
    for d in range(_PRE):
        wait_raw(d)
        build_idx(d)
        start_gathers(d)

    def group(tt, carry):
        for b in range(_NBUF):
            g = tt * _NBUF + b
            h = g + _PRE
            bh = (b + _PRE) % _NBUF

            @pl.when(h < _NCHUNK)
            def _():
                wait_raw(bh)
                build_idx(bh)
                start_gathers(bh)

            wait_gathers(b)

            def rowcp(r, carry):
                tok = lax.shift_right_logical(r, 1)
                cb = lax.bitwise_and(r, 1) * _D
                for k in range(3):
                    b50[tok, pl.ds(cb + 16 * k, 16)] = (
                        b48[b][r, pl.ds(16 * k, 16)])
                return carry

            lax.fori_loop(0, _CH, rowcp, 0)
            tail_fix(b)
            pltpu.sync_copy(b50, out_hbm.at[row0 + g])

            @pl.when(g + _NBUF < _NCHUNK)
            def _():
                fetch_raw(g + _NBUF, b)
        return carry

    lax.fori_loop(0, _NCHUNK // _NBUF, group, 0)


def kernel(words, pos_tags, word_table, pos_table):
    t48 = jnp.concatenate([word_table[:, :48], pos_table[:, :48]], axis=0)
    wf8 = word_table.reshape(_VOCAB * _D // 8, 8)
    return _emb_kernel(t48, wf8, words.astype(jnp.int32),
                       pos_tags.astype(jnp.int32), pos_table)
